# baseline (speedup 1.0000x reference)
import functools

import jax
import jax.numpy as jnp
from jax import lax
from jax.experimental import pallas as pl
from jax.experimental.pallas import tpu as pltpu
from jax.experimental.pallas import tpu_sc as plsc

N = 10000
E = 320000
F = 128
HF = F // 2
NS = 16
PE_T = 20480
CHUNKS = 160
C = 128
PE = NS * PE_T
DUMP = N
AGGR = 10240
AGGR_T = AGGR // NS
DEGT = 10240
DEGT_T = DEGT // NS

_mesh = plsc.VectorSubcoreMesh(core_axis_name="c", subcore_axis_name="s")


@functools.partial(
    pl.kernel,
    mesh=_mesh,
    compiler_params=pltpu.CompilerParams(use_tc_tiling_on_sc=False),
    out_type=jax.ShapeDtypeStruct((2, 2, DEGT, 16), jnp.float32),
    scratch_types=[
        pltpu.VMEM((CHUNKS, C), jnp.int32),
        pltpu.VMEM((CHUNKS, C), jnp.int32),
        pltpu.VMEM((C, 16), jnp.float32),
        pltpu.VMEM_SHARED((DEGT, 16), jnp.float32),
        pltpu.VMEM_SHARED((DEGT, 16), jnp.float32),
    ],
)
def _deg_kernel(srcd_h, dstd_h, z_h, ones_h, out_h,
                sidx_v, didx_v, ones_v, spo, spi):
    c = lax.axis_index("c")
    s = lax.axis_index("s")
    pltpu.sync_copy(srcd_h.at[c, s], sidx_v)
    pltpu.sync_copy(dstd_h.at[c, s], didx_v)
    pltpu.sync_copy(ones_h, ones_v)
    pltpu.sync_copy(z_h, spo.at[pl.ds(s * DEGT_T, DEGT_T)])
    pltpu.sync_copy(z_h, spi.at[pl.ds(s * DEGT_T, DEGT_T)])
    plsc.subcore_barrier()

    def body(g, carry):
        pltpu.sync_copy(ones_v, spo.at[sidx_v.at[g]], add=True)
        pltpu.sync_copy(ones_v, spi.at[didx_v.at[g]], add=True)
        return carry

    lax.fori_loop(0, CHUNKS, body, 0)
    plsc.subcore_barrier()
    pltpu.sync_copy(spo.at[pl.ds(s * DEGT_T, DEGT_T)],
                    out_h.at[c, 0, pl.ds(s * DEGT_T, DEGT_T)])
    pltpu.sync_copy(spi.at[pl.ds(s * DEGT_T, DEGT_T)],
                    out_h.at[c, 1, pl.ds(s * DEGT_T, DEGT_T)])


@functools.partial(
    pl.kernel,
    mesh=_mesh,
    compiler_params=pltpu.CompilerParams(use_tc_tiling_on_sc=False),
    out_type=[jax.ShapeDtypeStruct((2, AGGR, HF), jnp.float32),
              jax.ShapeDtypeStruct((2, AGGR, HF), jnp.float32)],
    scratch_types=[
        pltpu.VMEM((CHUNKS, C), jnp.int32),
        pltpu.VMEM((CHUNKS, C), jnp.int32),
        pltpu.VMEM((C, HF), jnp.float32),
        pltpu.VMEM((C, HF), jnp.float32),
        pltpu.VMEM((C, HF), jnp.float32),
        pltpu.VMEM((C, HF), jnp.float32),
        pltpu.VMEM_SHARED((AGGR, HF), jnp.float32),
        pltpu.SemaphoreType.DMA,
        pltpu.SemaphoreType.DMA,
        pltpu.SemaphoreType.DMA,
        pltpu.SemaphoreType.DMA,
        pltpu.SemaphoreType.DMA,
        pltpu.SemaphoreType.DMA,
        pltpu.SemaphoreType.DMA,
        pltpu.SemaphoreType.DMA,
    ],
)
def _agg_kernel(hs_lo_h, hs_hi_h, srcx_h, dstl_h, z_h, agg_lo_h, agg_hi_h,
                sidx_v, didx_v, buf0, buf1, buf2, buf3, aggsp,
                gs0, gs1, gs2, gs3, ss0, ss1, ss2, ss3):
    c = lax.axis_index("c")
    s = lax.axis_index("s")
    pltpu.sync_copy(srcx_h.at[c, s], sidx_v)
    pltpu.sync_copy(dstl_h.at[c, s], didx_v)
    slab = pl.ds(s * AGGR_T, AGGR_T)
    bufs = (buf0, buf1, buf2, buf3)
    gsems = (gs0, gs1, gs2, gs3)
    ssems = (ss0, ss1, ss2, ss3)
    NB = 4
    R = CHUNKS // NB

    def wait_gather(b, g):
        pltpu.make_async_copy(hs_h.at[sidx_v.at[g]], bufs[b], gsems[b]).wait()

    def wait_scatter(b, g):
        pltpu.make_async_copy(bufs[b], aggsp.at[didx_v.at[g]], ssems[b]).wait()

    for half in range(2):
        hs_h = (hs_lo_h, hs_hi_h)[half]
        agg_h = (agg_lo_h, agg_hi_h)[half]
        pltpu.sync_copy(z_h, aggsp.at[slab])
        plsc.subcore_barrier()

        for b in range(NB):
            pltpu.async_copy(hs_h.at[sidx_v.at[b]], bufs[b], gsems[b])

        def round_body(r, carry):
            g0 = r * NB
            for b in range(NB):
                wait_gather(b, g0 + b)
                pltpu.async_copy(bufs[b], aggsp.at[didx_v.at[g0 + b]],
                                 ssems[b], add=True)
            for b in range(NB):
                wait_scatter(b, g0 + b)
                pltpu.async_copy(hs_h.at[sidx_v.at[g0 + NB + b]], bufs[b],
                                 gsems[b])
            return carry

        lax.fori_loop(0, R - 1, round_body, 0)
        g0 = (R - 1) * NB
        for b in range(NB):
            wait_gather(b, g0 + b)
            pltpu.async_copy(bufs[b], aggsp.at[didx_v.at[g0 + b]],
                             ssems[b], add=True)
        for b in range(NB):
            wait_scatter(b, g0 + b)

        plsc.subcore_barrier()
        pltpu.sync_copy(aggsp.at[slab], agg_h.at[c, slab])


def _prep1_body(x_ref, dego_ref, w_ref, lo_ref, hi_ref):
    d = dego_ref[...]
    ns = jnp.where(d > 0, lax.rsqrt(jnp.maximum(d, 1e-9)), 0.0)
    h = jnp.dot(x_ref[...], w_ref[...], preferred_element_type=jnp.float32)
    h = h * ns
    lo_ref[...] = h[:, :HF]
    hi_ref[...] = h[:, HF:]


def _prep1(x, dego, w1):
    blk = 2000
    return pl.pallas_call(
        _prep1_body,
        grid=(2 * N // blk,),
        in_specs=[
            pl.BlockSpec((blk, F), lambda i: (i, 0)),
            pl.BlockSpec((blk, 1), lambda i: (i, 0)),
            pl.BlockSpec((F, F), lambda i: (0, 0)),
        ],
        out_specs=[pl.BlockSpec((blk, HF), lambda i: (i, 0)),
                   pl.BlockSpec((blk, HF), lambda i: (i, 0))],
        out_shape=[jax.ShapeDtypeStruct((2 * N, HF), jnp.float32),
                   jax.ShapeDtypeStruct((2 * N, HF), jnp.float32)],
    )(x, dego, w1)


def _prep2_body(alo_ref, ahi_ref, degi_ref, dego_ref, b1_ref, w2_ref,
                lo_ref, hi_ref):
    di = degi_ref[...]
    nd = jnp.where(di > 0, lax.rsqrt(jnp.maximum(di, 1e-9)), 0.0)
    do = dego_ref[...]
    ns = jnp.where(do > 0, lax.rsqrt(jnp.maximum(do, 1e-9)), 0.0)
    agg = jnp.concatenate([alo_ref[...], ahi_ref[...]], axis=-1)
    h = jnp.maximum(agg * nd + b1_ref[...], 0.0)
    h = jnp.dot(h, w2_ref[...], preferred_element_type=jnp.float32) * ns
    lo_ref[...] = h[:, :HF]
    hi_ref[...] = h[:, HF:]


def _prep2(alo, ahi, degi, dego, b1, w2):
    blk = 2000
    return pl.pallas_call(
        _prep2_body,
        grid=(2 * N // blk,),
        in_specs=[
            pl.BlockSpec((None, blk, HF), lambda i: (i // 5, i % 5, 0)),
            pl.BlockSpec((None, blk, HF), lambda i: (i // 5, i % 5, 0)),
            pl.BlockSpec((blk, 1), lambda i: (i, 0)),
            pl.BlockSpec((blk, 1), lambda i: (i, 0)),
            pl.BlockSpec((1, F), lambda i: (0, 0)),
            pl.BlockSpec((F, F), lambda i: (0, 0)),
        ],
        out_specs=[pl.BlockSpec((blk, HF), lambda i: (i, 0)),
                   pl.BlockSpec((blk, HF), lambda i: (i, 0))],
        out_shape=[jax.ShapeDtypeStruct((2 * N, HF), jnp.float32),
                   jax.ShapeDtypeStruct((2 * N, HF), jnp.float32)],
    )(alo, ahi, degi, dego, b1, w2)


def _final_body(alo_ref, ahi_ref, degi_ref, b2_ref, lw1_ref, lb1_ref,
                lw2_ref, lb2_ref, lw3_ref, lb3_ref, o_ref, acc_ref):
    i = pl.program_id(0)
    nblk = pl.num_programs(0)

    @pl.when(i == 0)
    def _init():
        acc_ref[...] = jnp.zeros_like(acc_ref)

    di = degi_ref[...]
    nd = jnp.where(di > 0, lax.rsqrt(jnp.maximum(di, 1e-9)), 0.0)
    agg = jnp.concatenate([alo_ref[...], ahi_ref[...]], axis=-1)
    h = jnp.maximum(agg * nd + b2_ref[...], 0.0)
    part = jnp.sum(h, axis=0, keepdims=True)
    g = i // (nblk // 2)
    rowmask = (lax.broadcasted_iota(jnp.int32, (8, 1), 0) == g)
    acc_ref[...] += jnp.where(rowmask, part, 0.0)

    @pl.when(i == nblk - 1)
    def _head():
        m1 = acc_ref[0:1, :] * (1.0 / N)
        m2 = acc_ref[1:2, :] * (1.0 / N)
        hg = m1 * m2
        l1 = jnp.dot(hg, lw1_ref[...], preferred_element_type=jnp.float32) + lb1_ref[...]
        l2 = jnp.dot(l1, lw2_ref[...], preferred_element_type=jnp.float32) + lb2_ref[...]
        l3 = jnp.dot(l2, lw3_ref[...], preferred_element_type=jnp.float32) + lb3_ref[...]
        o_ref[...] = jax.nn.softmax(l3, axis=-1)


def _final(alo, ahi, degi, b2, lw1, lb1, lw2, lb2, lw3, lb3):
    blk = 2000
    return pl.pallas_call(
        _final_body,
        grid=(2 * N // blk,),
        in_specs=[
            pl.BlockSpec((None, blk, HF), lambda i: (i // 5, i % 5, 0)),
            pl.BlockSpec((None, blk, HF), lambda i: (i // 5, i % 5, 0)),
            pl.BlockSpec((blk, 1), lambda i: (i, 0)),
            pl.BlockSpec((1, F), lambda i: (0, 0)),
            pl.BlockSpec((F, 512), lambda i: (0, 0)),
            pl.BlockSpec((1, 512), lambda i: (0, 0)),
            pl.BlockSpec((512, F), lambda i: (0, 0)),
            pl.BlockSpec((1, F), lambda i: (0, 0)),
            pl.BlockSpec((F, 2), lambda i: (0, 0)),
            pl.BlockSpec((1, 2), lambda i: (0, 0)),
        ],
        out_specs=pl.BlockSpec((1, 2), lambda i: (0, 0)),
        out_shape=jax.ShapeDtypeStruct((1, 2), jnp.float32),
        scratch_shapes=[pltpu.VMEM((8, F), jnp.float32)],
    )(alo, ahi, degi, b2, lw1, lb1, lw2, lb2, lw3, lb3)


def _padv(a, v):
    return jnp.concatenate([a, jnp.full((PE - E,), v, jnp.int32)])


def kernel(fea1, fea2, edge_index1, edge_index2, W1, b1, W2, b2,
           lw1, lb1, lw2, lb2, lw3, lb3):
    s1, d1 = edge_index1[0], edge_index1[1]
    s2, d2 = edge_index2[0], edge_index2[1]

    srcd = jnp.stack([_padv(s1, DUMP), _padv(s2, DUMP)]).reshape(2, NS, CHUNKS, C)
    srcx = jnp.stack([_padv(s1, 0), _padv(s2, 0) + N]).reshape(2, NS, CHUNKS, C)
    dstl = jnp.stack([_padv(d1, DUMP), _padv(d2, DUMP)]).reshape(2, NS, CHUNKS, C)

    zdeg = jnp.zeros((DEGT_T, 16), jnp.float32)
    ones = jnp.ones((C, 16), jnp.float32)
    zagg = jnp.zeros((AGGR_T, HF), jnp.float32)

    degs = _deg_kernel(srcd, dstl, zdeg, ones)
    deg = degs[:, :, :N, 0]
    dego = jnp.concatenate([deg[0, 0], deg[1, 0]]).reshape(2 * N, 1)
    degi = jnp.concatenate([deg[0, 1], deg[1, 1]]).reshape(2 * N, 1)

    x = jnp.concatenate([fea1, fea2], axis=0)
    hs1_lo, hs1_hi = _prep1(x, dego, W1)
    a1_lo, a1_hi = _agg_kernel(hs1_lo, hs1_hi, srcx, dstl, zagg)
    hs2_lo, hs2_hi = _prep2(a1_lo, a1_hi, degi, dego, b1.reshape(1, F), W2)
    a2_lo, a2_hi = _agg_kernel(hs2_lo, hs2_hi, srcx, dstl, zagg)
    return _final(a2_lo, a2_hi, degi, b2.reshape(1, F), lw1,
                  lb1.reshape(1, 512), lw2, lb2.reshape(1, F),
                  lw3, lb3.reshape(1, 2))


# speedup vs baseline: 1.5792x; 1.5792x over previous
import functools

import jax
import jax.numpy as jnp
from jax import lax
from jax.experimental import pallas as pl
from jax.experimental.pallas import tpu as pltpu
from jax.experimental.pallas import tpu_sc as plsc

N = 10000
E = 320000
F = 128
NS = 16
PE_T = 20480
CHUNKS = 160
C = 128
PE = NS * PE_T
DUMP = N
AGGR = 10240
AGGR_T = AGGR // NS
DEGT = 10240
DEGT_T = DEGT // NS

_mesh = plsc.VectorSubcoreMesh(core_axis_name="c", subcore_axis_name="s")


@functools.partial(
    pl.kernel,
    mesh=_mesh,
    compiler_params=pltpu.CompilerParams(use_tc_tiling_on_sc=False),
    out_type=jax.ShapeDtypeStruct((2, 2, DEGT, 16), jnp.float32),
    scratch_types=[
        pltpu.VMEM((CHUNKS, C), jnp.int32),
        pltpu.VMEM((CHUNKS, C), jnp.int32),
        pltpu.VMEM((C, 16), jnp.float32),
        pltpu.VMEM_SHARED((DEGT, 16), jnp.float32),
        pltpu.VMEM_SHARED((DEGT, 16), jnp.float32),
    ],
)
def _deg_kernel(srcd_h, dstd_h, z_h, ones_h, out_h,
                sidx_v, didx_v, ones_v, spo, spi):
    c = lax.axis_index("c")
    s = lax.axis_index("s")
    pltpu.sync_copy(srcd_h.at[c, s], sidx_v)
    pltpu.sync_copy(dstd_h.at[c, s], didx_v)
    pltpu.sync_copy(ones_h, ones_v)
    pltpu.sync_copy(z_h, spo.at[pl.ds(s * DEGT_T, DEGT_T)])
    pltpu.sync_copy(z_h, spi.at[pl.ds(s * DEGT_T, DEGT_T)])
    plsc.subcore_barrier()

    def body(g, carry):
        pltpu.sync_copy(ones_v, spo.at[sidx_v.at[g]], add=True)
        pltpu.sync_copy(ones_v, spi.at[didx_v.at[g]], add=True)
        return carry

    lax.fori_loop(0, CHUNKS, body, 0)
    plsc.subcore_barrier()
    pltpu.sync_copy(spo.at[pl.ds(s * DEGT_T, DEGT_T)],
                    out_h.at[c, 0, pl.ds(s * DEGT_T, DEGT_T)])
    pltpu.sync_copy(spi.at[pl.ds(s * DEGT_T, DEGT_T)],
                    out_h.at[c, 1, pl.ds(s * DEGT_T, DEGT_T)])


@functools.partial(
    pl.kernel,
    mesh=_mesh,
    compiler_params=pltpu.CompilerParams(use_tc_tiling_on_sc=False),
    out_type=jax.ShapeDtypeStruct((2, AGGR, F), jnp.bfloat16),
    scratch_types=[
        pltpu.VMEM((CHUNKS, C), jnp.int32),
        pltpu.VMEM((CHUNKS, C), jnp.int32),
        pltpu.VMEM((C, F), jnp.bfloat16),
        pltpu.VMEM((C, F), jnp.bfloat16),
        pltpu.VMEM((C, F), jnp.bfloat16),
        pltpu.VMEM((C, F), jnp.bfloat16),
        pltpu.VMEM_SHARED((AGGR, F), jnp.bfloat16),
        pltpu.SemaphoreType.DMA,
        pltpu.SemaphoreType.DMA,
        pltpu.SemaphoreType.DMA,
        pltpu.SemaphoreType.DMA,
        pltpu.SemaphoreType.DMA,
        pltpu.SemaphoreType.DMA,
        pltpu.SemaphoreType.DMA,
        pltpu.SemaphoreType.DMA,
    ],
)
def _agg_kernel(hs_h, srcx_h, dstl_h, z_h, agg_h,
                sidx_v, didx_v, buf0, buf1, buf2, buf3, aggsp,
                gs0, gs1, gs2, gs3, ss0, ss1, ss2, ss3):
    c = lax.axis_index("c")
    s = lax.axis_index("s")
    pltpu.sync_copy(srcx_h.at[c, s], sidx_v)
    pltpu.sync_copy(dstl_h.at[c, s], didx_v)
    slab = pl.ds(s * AGGR_T, AGGR_T)
    bufs = (buf0, buf1, buf2, buf3)
    gsems = (gs0, gs1, gs2, gs3)
    ssems = (ss0, ss1, ss2, ss3)
    NB = 4
    R = CHUNKS // NB

    def wait_gather(b, g):
        pltpu.make_async_copy(hs_h.at[sidx_v.at[g]], bufs[b], gsems[b]).wait()

    def wait_scatter(b, g):
        pltpu.make_async_copy(bufs[b], aggsp.at[didx_v.at[g]], ssems[b]).wait()

    pltpu.sync_copy(z_h, aggsp.at[slab])
    plsc.subcore_barrier()

    for b in range(NB):
        pltpu.async_copy(hs_h.at[sidx_v.at[b]], bufs[b], gsems[b])

    def round_body(r, carry):
        g0 = r * NB
        for b in range(NB):
            wait_gather(b, g0 + b)
            pltpu.async_copy(bufs[b], aggsp.at[didx_v.at[g0 + b]],
                             ssems[b], add=True)
        for b in range(NB):
            wait_scatter(b, g0 + b)
            pltpu.async_copy(hs_h.at[sidx_v.at[g0 + NB + b]], bufs[b],
                             gsems[b])
        return carry

    lax.fori_loop(0, R - 1, round_body, 0)
    g0 = (R - 1) * NB
    for b in range(NB):
        wait_gather(b, g0 + b)
        pltpu.async_copy(bufs[b], aggsp.at[didx_v.at[g0 + b]],
                         ssems[b], add=True)
    for b in range(NB):
        wait_scatter(b, g0 + b)

    plsc.subcore_barrier()
    pltpu.sync_copy(aggsp.at[slab], agg_h.at[c, slab])


def _prep1_body(x_ref, dego_ref, w_ref, o_ref):
    d = dego_ref[...]
    ns = jnp.where(d > 0, lax.rsqrt(jnp.maximum(d, 1e-9)), 0.0)
    h = jnp.dot(x_ref[...], w_ref[...], preferred_element_type=jnp.float32)
    o_ref[...] = (h * ns).astype(jnp.bfloat16)


def _prep1(x, dego, w1):
    blk = 2000
    return pl.pallas_call(
        _prep1_body,
        grid=(2 * N // blk,),
        in_specs=[
            pl.BlockSpec((blk, F), lambda i: (i, 0)),
            pl.BlockSpec((blk, 1), lambda i: (i, 0)),
            pl.BlockSpec((F, F), lambda i: (0, 0)),
        ],
        out_specs=pl.BlockSpec((blk, F), lambda i: (i, 0)),
        out_shape=jax.ShapeDtypeStruct((2 * N, F), jnp.bfloat16),
    )(x, dego, w1)


def _prep2_body(agg_ref, degi_ref, dego_ref, b1_ref, w2_ref, o_ref):
    di = degi_ref[...]
    nd = jnp.where(di > 0, lax.rsqrt(jnp.maximum(di, 1e-9)), 0.0)
    do = dego_ref[...]
    ns = jnp.where(do > 0, lax.rsqrt(jnp.maximum(do, 1e-9)), 0.0)
    agg = agg_ref[...].astype(jnp.float32)
    h = jnp.maximum(agg * nd + b1_ref[...], 0.0)
    h = jnp.dot(h, w2_ref[...], preferred_element_type=jnp.float32) * ns
    o_ref[...] = h.astype(jnp.bfloat16)


def _prep2(agg, degi, dego, b1, w2):
    blk = 2000
    return pl.pallas_call(
        _prep2_body,
        grid=(2 * N // blk,),
        in_specs=[
            pl.BlockSpec((None, blk, F), lambda i: (i // 5, i % 5, 0)),
            pl.BlockSpec((blk, 1), lambda i: (i, 0)),
            pl.BlockSpec((blk, 1), lambda i: (i, 0)),
            pl.BlockSpec((1, F), lambda i: (0, 0)),
            pl.BlockSpec((F, F), lambda i: (0, 0)),
        ],
        out_specs=pl.BlockSpec((blk, F), lambda i: (i, 0)),
        out_shape=jax.ShapeDtypeStruct((2 * N, F), jnp.bfloat16),
    )(agg, degi, dego, b1, w2)


def _final_body(agg_ref, degi_ref, b2_ref, lw1_ref, lb1_ref,
                lw2_ref, lb2_ref, lw3_ref, lb3_ref, o_ref, acc_ref):
    i = pl.program_id(0)
    nblk = pl.num_programs(0)

    @pl.when(i == 0)
    def _init():
        acc_ref[...] = jnp.zeros_like(acc_ref)

    di = degi_ref[...]
    nd = jnp.where(di > 0, lax.rsqrt(jnp.maximum(di, 1e-9)), 0.0)
    agg = agg_ref[...].astype(jnp.float32)
    h = jnp.maximum(agg * nd + b2_ref[...], 0.0)
    part = jnp.sum(h, axis=0, keepdims=True)
    g = i // (nblk // 2)
    rowmask = (lax.broadcasted_iota(jnp.int32, (8, 1), 0) == g)
    acc_ref[...] += jnp.where(rowmask, part, 0.0)

    @pl.when(i == nblk - 1)
    def _head():
        m1 = acc_ref[0:1, :] * (1.0 / N)
        m2 = acc_ref[1:2, :] * (1.0 / N)
        hg = m1 * m2
        l1 = jnp.dot(hg, lw1_ref[...], preferred_element_type=jnp.float32) + lb1_ref[...]
        l2 = jnp.dot(l1, lw2_ref[...], preferred_element_type=jnp.float32) + lb2_ref[...]
        l3 = jnp.dot(l2, lw3_ref[...], preferred_element_type=jnp.float32) + lb3_ref[...]
        o_ref[...] = jax.nn.softmax(l3, axis=-1)


def _final(agg, degi, b2, lw1, lb1, lw2, lb2, lw3, lb3):
    blk = 2000
    return pl.pallas_call(
        _final_body,
        grid=(2 * N // blk,),
        in_specs=[
            pl.BlockSpec((None, blk, F), lambda i: (i // 5, i % 5, 0)),
            pl.BlockSpec((blk, 1), lambda i: (i, 0)),
            pl.BlockSpec((1, F), lambda i: (0, 0)),
            pl.BlockSpec((F, 512), lambda i: (0, 0)),
            pl.BlockSpec((1, 512), lambda i: (0, 0)),
            pl.BlockSpec((512, F), lambda i: (0, 0)),
            pl.BlockSpec((1, F), lambda i: (0, 0)),
            pl.BlockSpec((F, 2), lambda i: (0, 0)),
            pl.BlockSpec((1, 2), lambda i: (0, 0)),
        ],
        out_specs=pl.BlockSpec((1, 2), lambda i: (0, 0)),
        out_shape=jax.ShapeDtypeStruct((1, 2), jnp.float32),
        scratch_shapes=[pltpu.VMEM((8, F), jnp.float32)],
    )(agg, degi, b2, lw1, lb1, lw2, lb2, lw3, lb3)


def _padv(a, v):
    return jnp.concatenate([a, jnp.full((PE - E,), v, jnp.int32)])


def kernel(fea1, fea2, edge_index1, edge_index2, W1, b1, W2, b2,
           lw1, lb1, lw2, lb2, lw3, lb3):
    s1, d1 = edge_index1[0], edge_index1[1]
    s2, d2 = edge_index2[0], edge_index2[1]

    srcd = jnp.stack([_padv(s1, DUMP), _padv(s2, DUMP)]).reshape(2, NS, CHUNKS, C)
    srcx = jnp.stack([_padv(s1, 0), _padv(s2, 0) + N]).reshape(2, NS, CHUNKS, C)
    dstl = jnp.stack([_padv(d1, DUMP), _padv(d2, DUMP)]).reshape(2, NS, CHUNKS, C)

    zdeg = jnp.zeros((DEGT_T, 16), jnp.float32)
    ones = jnp.ones((C, 16), jnp.float32)
    zagg = jnp.zeros((AGGR_T, F), jnp.bfloat16)

    degs = _deg_kernel(srcd, dstl, zdeg, ones)
    deg = degs[:, :, :N, 0]
    dego = jnp.concatenate([deg[0, 0], deg[1, 0]]).reshape(2 * N, 1)
    degi = jnp.concatenate([deg[0, 1], deg[1, 1]]).reshape(2 * N, 1)

    x = jnp.concatenate([fea1, fea2], axis=0)
    hs1 = _prep1(x, dego, W1)
    a1 = _agg_kernel(hs1, srcx, dstl, zagg)
    hs2 = _prep2(a1, degi, dego, b1.reshape(1, F), W2)
    a2 = _agg_kernel(hs2, srcx, dstl, zagg)
    return _final(a2, degi, b2.reshape(1, F), lw1,
                  lb1.reshape(1, 512), lw2, lb2.reshape(1, F),
                  lw3, lb3.reshape(1, 2))


# speedup vs baseline: 1.5982x; 1.0121x over previous
import functools

import jax
import jax.numpy as jnp
from jax import lax
from jax.experimental import pallas as pl
from jax.experimental.pallas import tpu as pltpu
from jax.experimental.pallas import tpu_sc as plsc

N = 10000
E = 320000
F = 128
NS = 16
PE_T = 20480
CHUNKS = 160
C = 128
PE = NS * PE_T
DUMP = N
AGGR = 10240
AGGR_T = AGGR // NS
DEGT = 10240
DEGT_T = DEGT // NS

_mesh = plsc.VectorSubcoreMesh(core_axis_name="c", subcore_axis_name="s")


@functools.partial(
    pl.kernel,
    mesh=_mesh,
    compiler_params=pltpu.CompilerParams(use_tc_tiling_on_sc=False),
    out_type=jax.ShapeDtypeStruct((2, 2, DEGT, 16), jnp.float32),
    scratch_types=[
        pltpu.VMEM((CHUNKS, C), jnp.int32),
        pltpu.VMEM((CHUNKS, C), jnp.int32),
        pltpu.VMEM((C, 16), jnp.float32),
        pltpu.VMEM_SHARED((DEGT, 16), jnp.float32),
        pltpu.VMEM_SHARED((DEGT, 16), jnp.float32),
        pltpu.SemaphoreType.DMA,
        pltpu.SemaphoreType.DMA,
    ],
)
def _deg_kernel(srcd_h, dstd_h, z_h, ones_h, out_h,
                sidx_v, didx_v, ones_v, spo, spi, sso, ssi):
    c = lax.axis_index("c")
    s = lax.axis_index("s")
    pltpu.sync_copy(srcd_h.at[c, s], sidx_v)
    pltpu.sync_copy(dstd_h.at[c, s], didx_v)
    pltpu.sync_copy(ones_h, ones_v)
    pltpu.sync_copy(z_h, spo.at[pl.ds(s * DEGT_T, DEGT_T)])
    pltpu.sync_copy(z_h, spi.at[pl.ds(s * DEGT_T, DEGT_T)])
    plsc.subcore_barrier()

    K = 4
    for k in range(K):
        pltpu.async_copy(ones_v, spo.at[sidx_v.at[k]], sso, add=True)
        pltpu.async_copy(ones_v, spi.at[didx_v.at[k]], ssi, add=True)

    def body(g, carry):
        pltpu.make_async_copy(ones_v, spo.at[sidx_v.at[g]], sso).wait()
        pltpu.make_async_copy(ones_v, spi.at[didx_v.at[g]], ssi).wait()
        pltpu.async_copy(ones_v, spo.at[sidx_v.at[g + K]], sso, add=True)
        pltpu.async_copy(ones_v, spi.at[didx_v.at[g + K]], ssi, add=True)
        return carry

    lax.fori_loop(0, CHUNKS - K, body, 0)
    for g in range(CHUNKS - K, CHUNKS):
        pltpu.make_async_copy(ones_v, spo.at[sidx_v.at[g]], sso).wait()
        pltpu.make_async_copy(ones_v, spi.at[didx_v.at[g]], ssi).wait()
    plsc.subcore_barrier()
    pltpu.sync_copy(spo.at[pl.ds(s * DEGT_T, DEGT_T)],
                    out_h.at[c, 0, pl.ds(s * DEGT_T, DEGT_T)])
    pltpu.sync_copy(spi.at[pl.ds(s * DEGT_T, DEGT_T)],
                    out_h.at[c, 1, pl.ds(s * DEGT_T, DEGT_T)])


@functools.partial(
    pl.kernel,
    mesh=_mesh,
    compiler_params=pltpu.CompilerParams(use_tc_tiling_on_sc=False),
    out_type=jax.ShapeDtypeStruct((2, AGGR, F), jnp.bfloat16),
    scratch_types=[
        pltpu.VMEM((CHUNKS, C), jnp.int32),
        pltpu.VMEM((CHUNKS, C), jnp.int32),
        pltpu.VMEM((C, F), jnp.bfloat16),
        pltpu.VMEM((C, F), jnp.bfloat16),
        pltpu.VMEM((C, F), jnp.bfloat16),
        pltpu.VMEM((C, F), jnp.bfloat16),
        pltpu.VMEM_SHARED((AGGR, F), jnp.bfloat16),
        pltpu.SemaphoreType.DMA,
        pltpu.SemaphoreType.DMA,
        pltpu.SemaphoreType.DMA,
        pltpu.SemaphoreType.DMA,
        pltpu.SemaphoreType.DMA,
        pltpu.SemaphoreType.DMA,
        pltpu.SemaphoreType.DMA,
        pltpu.SemaphoreType.DMA,
    ],
)
def _agg_kernel(hs_h, srcx_h, dstl_h, z_h, agg_h,
                sidx_v, didx_v, buf0, buf1, buf2, buf3, aggsp,
                gs0, gs1, gs2, gs3, ss0, ss1, ss2, ss3):
    c = lax.axis_index("c")
    s = lax.axis_index("s")
    pltpu.sync_copy(srcx_h.at[c, s], sidx_v)
    pltpu.sync_copy(dstl_h.at[c, s], didx_v)
    slab = pl.ds(s * AGGR_T, AGGR_T)
    bufs = (buf0, buf1, buf2, buf3)
    gsems = (gs0, gs1, gs2, gs3)
    ssems = (ss0, ss1, ss2, ss3)
    NB = 4
    R = CHUNKS // NB

    def wait_gather(b, g):
        pltpu.make_async_copy(hs_h.at[sidx_v.at[g]], bufs[b], gsems[b]).wait()

    def wait_scatter(b, g):
        pltpu.make_async_copy(bufs[b], aggsp.at[didx_v.at[g]], ssems[b]).wait()

    pltpu.sync_copy(z_h, aggsp.at[slab])
    plsc.subcore_barrier()

    for b in range(NB):
        pltpu.async_copy(hs_h.at[sidx_v.at[b]], bufs[b], gsems[b])

    def round_body(r, carry):
        g0 = r * NB
        for b in range(NB):
            wait_gather(b, g0 + b)
            pltpu.async_copy(bufs[b], aggsp.at[didx_v.at[g0 + b]],
                             ssems[b], add=True)
        for b in range(NB):
            wait_scatter(b, g0 + b)
            pltpu.async_copy(hs_h.at[sidx_v.at[g0 + NB + b]], bufs[b],
                             gsems[b])
        return carry

    lax.fori_loop(0, R - 1, round_body, 0)
    g0 = (R - 1) * NB
    for b in range(NB):
        wait_gather(b, g0 + b)
        pltpu.async_copy(bufs[b], aggsp.at[didx_v.at[g0 + b]],
                         ssems[b], add=True)
    for b in range(NB):
        wait_scatter(b, g0 + b)

    plsc.subcore_barrier()
    pltpu.sync_copy(aggsp.at[slab], agg_h.at[c, slab])


def _mm1_body(x_ref, w_ref, o_ref):
    o_ref[...] = jnp.dot(x_ref[...], w_ref[...],
                         preferred_element_type=jnp.float32)


def _mm1(x, w1):
    blk = 2000
    return pl.pallas_call(
        _mm1_body,
        grid=(2 * N // blk,),
        in_specs=[
            pl.BlockSpec((blk, F), lambda i: (i, 0)),
            pl.BlockSpec((F, F), lambda i: (0, 0)),
        ],
        out_specs=pl.BlockSpec((blk, F), lambda i: (i, 0)),
        out_shape=jax.ShapeDtypeStruct((2 * N, F), jnp.float32),
    )(x, w1)


def _scale1_body(u_ref, dego_ref, o_ref):
    d = dego_ref[...]
    ns = jnp.where(d > 0, lax.rsqrt(jnp.maximum(d, 1e-9)), 0.0)
    o_ref[...] = (u_ref[...] * ns).astype(jnp.bfloat16)


def _scale1(u, dego):
    blk = 2000
    return pl.pallas_call(
        _scale1_body,
        grid=(2 * N // blk,),
        in_specs=[
            pl.BlockSpec((blk, F), lambda i: (i, 0)),
            pl.BlockSpec((blk, 1), lambda i: (i, 0)),
        ],
        out_specs=pl.BlockSpec((blk, F), lambda i: (i, 0)),
        out_shape=jax.ShapeDtypeStruct((2 * N, F), jnp.bfloat16),
    )(u, dego)


def _prep2_body(agg_ref, degi_ref, dego_ref, b1_ref, w2_ref, o_ref):
    di = degi_ref[...]
    nd = jnp.where(di > 0, lax.rsqrt(jnp.maximum(di, 1e-9)), 0.0)
    do = dego_ref[...]
    ns = jnp.where(do > 0, lax.rsqrt(jnp.maximum(do, 1e-9)), 0.0)
    agg = agg_ref[...].astype(jnp.float32)
    h = jnp.maximum(agg * nd + b1_ref[...], 0.0)
    h = jnp.dot(h, w2_ref[...], preferred_element_type=jnp.float32) * ns
    o_ref[...] = h.astype(jnp.bfloat16)


def _prep2(agg, degi, dego, b1, w2):
    blk = 2000
    return pl.pallas_call(
        _prep2_body,
        grid=(2 * N // blk,),
        in_specs=[
            pl.BlockSpec((None, blk, F), lambda i: (i // 5, i % 5, 0)),
            pl.BlockSpec((blk, 1), lambda i: (i, 0)),
            pl.BlockSpec((blk, 1), lambda i: (i, 0)),
            pl.BlockSpec((1, F), lambda i: (0, 0)),
            pl.BlockSpec((F, F), lambda i: (0, 0)),
        ],
        out_specs=pl.BlockSpec((blk, F), lambda i: (i, 0)),
        out_shape=jax.ShapeDtypeStruct((2 * N, F), jnp.bfloat16),
    )(agg, degi, dego, b1, w2)


def _final_body(agg_ref, degi_ref, b2_ref, lw1_ref, lb1_ref,
                lw2_ref, lb2_ref, lw3_ref, lb3_ref, o_ref, acc_ref):
    i = pl.program_id(0)
    nblk = pl.num_programs(0)

    @pl.when(i == 0)
    def _init():
        acc_ref[...] = jnp.zeros_like(acc_ref)

    di = degi_ref[...]
    nd = jnp.where(di > 0, lax.rsqrt(jnp.maximum(di, 1e-9)), 0.0)
    agg = agg_ref[...].astype(jnp.float32)
    h = jnp.maximum(agg * nd + b2_ref[...], 0.0)
    part = jnp.sum(h, axis=0, keepdims=True)
    g = i // (nblk // 2)
    rowmask = (lax.broadcasted_iota(jnp.int32, (8, 1), 0) == g)
    acc_ref[...] += jnp.where(rowmask, part, 0.0)

    @pl.when(i == nblk - 1)
    def _head():
        m1 = acc_ref[0:1, :] * (1.0 / N)
        m2 = acc_ref[1:2, :] * (1.0 / N)
        hg = m1 * m2
        l1 = jnp.dot(hg, lw1_ref[...], preferred_element_type=jnp.float32) + lb1_ref[...]
        l2 = jnp.dot(l1, lw2_ref[...], preferred_element_type=jnp.float32) + lb2_ref[...]
        l3 = jnp.dot(l2, lw3_ref[...], preferred_element_type=jnp.float32) + lb3_ref[...]
        o_ref[...] = jax.nn.softmax(l3, axis=-1)


def _final(agg, degi, b2, lw1, lb1, lw2, lb2, lw3, lb3):
    blk = 2000
    return pl.pallas_call(
        _final_body,
        grid=(2 * N // blk,),
        in_specs=[
            pl.BlockSpec((None, blk, F), lambda i: (i // 5, i % 5, 0)),
            pl.BlockSpec((blk, 1), lambda i: (i, 0)),
            pl.BlockSpec((1, F), lambda i: (0, 0)),
            pl.BlockSpec((F, 512), lambda i: (0, 0)),
            pl.BlockSpec((1, 512), lambda i: (0, 0)),
            pl.BlockSpec((512, F), lambda i: (0, 0)),
            pl.BlockSpec((1, F), lambda i: (0, 0)),
            pl.BlockSpec((F, 2), lambda i: (0, 0)),
            pl.BlockSpec((1, 2), lambda i: (0, 0)),
        ],
        out_specs=pl.BlockSpec((1, 2), lambda i: (0, 0)),
        out_shape=jax.ShapeDtypeStruct((1, 2), jnp.float32),
        scratch_shapes=[pltpu.VMEM((8, F), jnp.float32)],
    )(agg, degi, b2, lw1, lb1, lw2, lb2, lw3, lb3)


def _padv(a, v):
    return jnp.concatenate([a, jnp.full((PE - E,), v, jnp.int32)])


def kernel(fea1, fea2, edge_index1, edge_index2, W1, b1, W2, b2,
           lw1, lb1, lw2, lb2, lw3, lb3):
    s1, d1 = edge_index1[0], edge_index1[1]
    s2, d2 = edge_index2[0], edge_index2[1]

    srcd = jnp.stack([_padv(s1, DUMP), _padv(s2, DUMP)]).reshape(2, NS, CHUNKS, C)
    srcx = jnp.stack([_padv(s1, 0), _padv(s2, 0) + N]).reshape(2, NS, CHUNKS, C)
    dstl = jnp.stack([_padv(d1, DUMP), _padv(d2, DUMP)]).reshape(2, NS, CHUNKS, C)

    zdeg = jnp.zeros((DEGT_T, 16), jnp.float32)
    ones = jnp.ones((C, 16), jnp.float32)
    zagg = jnp.zeros((AGGR_T, F), jnp.bfloat16)

    degs = _deg_kernel(srcd, dstl, zdeg, ones)
    deg = degs[:, :, :N, 0]
    dego = jnp.concatenate([deg[0, 0], deg[1, 0]]).reshape(2 * N, 1)
    degi = jnp.concatenate([deg[0, 1], deg[1, 1]]).reshape(2 * N, 1)

    x = jnp.concatenate([fea1, fea2], axis=0)
    u1 = _mm1(x, W1)
    hs1 = _scale1(u1, dego)
    a1 = _agg_kernel(hs1, srcx, dstl, zagg)
    hs2 = _prep2(a1, degi, dego, b1.reshape(1, F), W2)
    a2 = _agg_kernel(hs2, srcx, dstl, zagg)
    return _final(a2, degi, b2.reshape(1, F), lw1,
                  lb1.reshape(1, 512), lw2, lb2.reshape(1, F),
                  lw3, lb3.reshape(1, 2))


# speedup vs baseline: 1.6546x; 1.0352x over previous
import functools

import jax
import jax.numpy as jnp
from jax import lax
from jax.experimental import pallas as pl
from jax.experimental.pallas import tpu as pltpu
from jax.experimental.pallas import tpu_sc as plsc

N = 10000
E = 320000
F = 128
NS = 16
PE_T = 20480
CHUNKS = 160
C = 128
PE = NS * PE_T
DUMP = N
AGGR = 10048
AGGR_T = AGGR // NS
DEGT = 10048
DEGT_T = DEGT // NS

_mesh = plsc.VectorSubcoreMesh(core_axis_name="c", subcore_axis_name="s")


@functools.partial(
    pl.kernel,
    mesh=_mesh,
    compiler_params=pltpu.CompilerParams(use_tc_tiling_on_sc=False),
    out_type=jax.ShapeDtypeStruct((2, DEGT, 16), jnp.float32),
    scratch_types=[
        pltpu.VMEM((CHUNKS, C), jnp.int32),
        pltpu.VMEM((CHUNKS, C), jnp.int32),
        pltpu.VMEM((C, 16), jnp.float32),
        pltpu.VMEM((C, 16), jnp.float32),
        pltpu.VMEM_SHARED((DEGT, 16), jnp.float32),
        pltpu.SemaphoreType.DMA,
        pltpu.SemaphoreType.DMA,
    ],
)
def _deg_kernel(srcd_h, dstd_h, z_h, ones_lo_h, ones_hi_h, out_h,
                sidx_v, didx_v, olo_v, ohi_v, spd, sso, ssi):
    c = lax.axis_index("c")
    s = lax.axis_index("s")
    pltpu.sync_copy(srcd_h.at[c, s], sidx_v)
    pltpu.sync_copy(dstd_h.at[c, s], didx_v)
    pltpu.sync_copy(ones_lo_h, olo_v)
    pltpu.sync_copy(ones_hi_h, ohi_v)
    pltpu.sync_copy(z_h, spd.at[pl.ds(s * DEGT_T, DEGT_T)])
    plsc.subcore_barrier()

    K = 4
    for k in range(K):
        pltpu.async_copy(olo_v, spd.at[sidx_v.at[k]], sso, add=True)
        pltpu.async_copy(ohi_v, spd.at[didx_v.at[k]], ssi, add=True)

    def body(g, carry):
        pltpu.make_async_copy(olo_v, spd.at[sidx_v.at[g]], sso).wait()
        pltpu.make_async_copy(ohi_v, spd.at[didx_v.at[g]], ssi).wait()
        pltpu.async_copy(olo_v, spd.at[sidx_v.at[g + K]], sso, add=True)
        pltpu.async_copy(ohi_v, spd.at[didx_v.at[g + K]], ssi, add=True)
        return carry

    lax.fori_loop(0, CHUNKS - K, body, 0)
    for g in range(CHUNKS - K, CHUNKS):
        pltpu.make_async_copy(olo_v, spd.at[sidx_v.at[g]], sso).wait()
        pltpu.make_async_copy(ohi_v, spd.at[didx_v.at[g]], ssi).wait()
    plsc.subcore_barrier()
    pltpu.sync_copy(spd.at[pl.ds(s * DEGT_T, DEGT_T)],
                    out_h.at[c, pl.ds(s * DEGT_T, DEGT_T)])


@functools.partial(
    pl.kernel,
    mesh=_mesh,
    compiler_params=pltpu.CompilerParams(use_tc_tiling_on_sc=False),
    out_type=jax.ShapeDtypeStruct((2, AGGR, F), jnp.bfloat16),
    scratch_types=[
        pltpu.VMEM((CHUNKS, C), jnp.int32),
        pltpu.VMEM((CHUNKS, C), jnp.int32),
        pltpu.VMEM((C, F), jnp.bfloat16),
        pltpu.VMEM((C, F), jnp.bfloat16),
        pltpu.VMEM((C, F), jnp.bfloat16),
        pltpu.VMEM((C, F), jnp.bfloat16),
        pltpu.VMEM_SHARED((AGGR, F), jnp.bfloat16),
        pltpu.SemaphoreType.DMA,
        pltpu.SemaphoreType.DMA,
        pltpu.SemaphoreType.DMA,
        pltpu.SemaphoreType.DMA,
        pltpu.SemaphoreType.DMA,
        pltpu.SemaphoreType.DMA,
        pltpu.SemaphoreType.DMA,
        pltpu.SemaphoreType.DMA,
    ],
)
def _agg_kernel(hs_h, srcx_h, dstl_h, z_h, agg_h,
                sidx_v, didx_v, buf0, buf1, buf2, buf3, aggsp,
                gs0, gs1, gs2, gs3, ss0, ss1, ss2, ss3):
    c = lax.axis_index("c")
    s = lax.axis_index("s")
    pltpu.sync_copy(srcx_h.at[c, s], sidx_v)
    pltpu.sync_copy(dstl_h.at[c, s], didx_v)
    slab = pl.ds(s * AGGR_T, AGGR_T)
    bufs = (buf0, buf1, buf2, buf3)
    gsems = (gs0, gs1, gs2, gs3)
    ssems = (ss0, ss1, ss2, ss3)
    NB = 4
    R = CHUNKS // NB

    def wait_gather(b, g):
        pltpu.make_async_copy(hs_h.at[sidx_v.at[g]], bufs[b], gsems[b]).wait()

    def wait_scatter(b, g):
        pltpu.make_async_copy(bufs[b], aggsp.at[didx_v.at[g]], ssems[b]).wait()

    pltpu.sync_copy(z_h, aggsp.at[slab])
    plsc.subcore_barrier()

    for b in range(NB):
        pltpu.async_copy(hs_h.at[sidx_v.at[b]], bufs[b], gsems[b])

    def round_body(r, carry):
        g0 = r * NB
        for b in range(NB):
            wait_gather(b, g0 + b)
            pltpu.async_copy(bufs[b], aggsp.at[didx_v.at[g0 + b]],
                             ssems[b], add=True)
        for b in range(NB):
            wait_scatter(b, g0 + b)
            pltpu.async_copy(hs_h.at[sidx_v.at[g0 + NB + b]], bufs[b],
                             gsems[b])
        return carry

    lax.fori_loop(0, R - 1, round_body, 0)
    g0 = (R - 1) * NB
    for b in range(NB):
        wait_gather(b, g0 + b)
        pltpu.async_copy(bufs[b], aggsp.at[didx_v.at[g0 + b]],
                         ssems[b], add=True)
    for b in range(NB):
        wait_scatter(b, g0 + b)

    plsc.subcore_barrier()
    pltpu.sync_copy(aggsp.at[slab], agg_h.at[c, slab])


def _mm1_body(x_ref, w_ref, o_ref):
    o_ref[...] = jnp.dot(x_ref[...], w_ref[...],
                         preferred_element_type=jnp.float32)


def _mm1(x, w1):
    blk = 2000
    return pl.pallas_call(
        _mm1_body,
        grid=(2 * N // blk,),
        in_specs=[
            pl.BlockSpec((blk, F), lambda i: (i, 0)),
            pl.BlockSpec((F, F), lambda i: (0, 0)),
        ],
        out_specs=pl.BlockSpec((blk, F), lambda i: (i, 0)),
        out_shape=jax.ShapeDtypeStruct((2 * N, F), jnp.float32),
    )(x, w1)


def _scale1_body(u_ref, dego_ref, o_ref):
    d = dego_ref[...]
    ns = jnp.where(d > 0, lax.rsqrt(jnp.maximum(d, 1e-9)), 0.0)
    o_ref[...] = (u_ref[...] * ns).astype(jnp.bfloat16)


def _scale1(u, dego):
    blk = 2000
    return pl.pallas_call(
        _scale1_body,
        grid=(2 * N // blk,),
        in_specs=[
            pl.BlockSpec((blk, F), lambda i: (i, 0)),
            pl.BlockSpec((blk, 1), lambda i: (i, 0)),
        ],
        out_specs=pl.BlockSpec((blk, F), lambda i: (i, 0)),
        out_shape=jax.ShapeDtypeStruct((2 * N, F), jnp.bfloat16),
    )(u, dego)


def _prep2_body(agg_ref, degi_ref, dego_ref, b1_ref, w2_ref, o_ref):
    di = degi_ref[...]
    nd = jnp.where(di > 0, lax.rsqrt(jnp.maximum(di, 1e-9)), 0.0)
    do = dego_ref[...]
    ns = jnp.where(do > 0, lax.rsqrt(jnp.maximum(do, 1e-9)), 0.0)
    agg = agg_ref[...].astype(jnp.float32)
    h = jnp.maximum(agg * nd + b1_ref[...], 0.0)
    h = jnp.dot(h, w2_ref[...], preferred_element_type=jnp.float32) * ns
    o_ref[...] = h.astype(jnp.bfloat16)


def _prep2(agg, degi, dego, b1, w2):
    blk = 2000
    return pl.pallas_call(
        _prep2_body,
        grid=(2 * N // blk,),
        in_specs=[
            pl.BlockSpec((None, blk, F), lambda i: (i // 5, i % 5, 0)),
            pl.BlockSpec((blk, 1), lambda i: (i, 0)),
            pl.BlockSpec((blk, 1), lambda i: (i, 0)),
            pl.BlockSpec((1, F), lambda i: (0, 0)),
            pl.BlockSpec((F, F), lambda i: (0, 0)),
        ],
        out_specs=pl.BlockSpec((blk, F), lambda i: (i, 0)),
        out_shape=jax.ShapeDtypeStruct((2 * N, F), jnp.bfloat16),
    )(agg, degi, dego, b1, w2)


def _final_body(agg_ref, degi_ref, b2_ref, lw1_ref, lb1_ref,
                lw2_ref, lb2_ref, lw3_ref, lb3_ref, o_ref, acc_ref):
    i = pl.program_id(0)
    nblk = pl.num_programs(0)

    @pl.when(i == 0)
    def _init():
        acc_ref[...] = jnp.zeros_like(acc_ref)

    di = degi_ref[...]
    nd = jnp.where(di > 0, lax.rsqrt(jnp.maximum(di, 1e-9)), 0.0)
    agg = agg_ref[...].astype(jnp.float32)
    h = jnp.maximum(agg * nd + b2_ref[...], 0.0)
    part = jnp.sum(h, axis=0, keepdims=True)
    g = i // (nblk // 2)
    rowmask = (lax.broadcasted_iota(jnp.int32, (8, 1), 0) == g)
    acc_ref[...] += jnp.where(rowmask, part, 0.0)

    @pl.when(i == nblk - 1)
    def _head():
        m1 = acc_ref[0:1, :] * (1.0 / N)
        m2 = acc_ref[1:2, :] * (1.0 / N)
        hg = m1 * m2
        l1 = jnp.dot(hg, lw1_ref[...], preferred_element_type=jnp.float32) + lb1_ref[...]
        l2 = jnp.dot(l1, lw2_ref[...], preferred_element_type=jnp.float32) + lb2_ref[...]
        l3 = jnp.dot(l2, lw3_ref[...], preferred_element_type=jnp.float32) + lb3_ref[...]
        o_ref[...] = jax.nn.softmax(l3, axis=-1)


def _final(agg, degi, b2, lw1, lb1, lw2, lb2, lw3, lb3):
    blk = 2000
    return pl.pallas_call(
        _final_body,
        grid=(2 * N // blk,),
        in_specs=[
            pl.BlockSpec((None, blk, F), lambda i: (i // 5, i % 5, 0)),
            pl.BlockSpec((blk, 1), lambda i: (i, 0)),
            pl.BlockSpec((1, F), lambda i: (0, 0)),
            pl.BlockSpec((F, 512), lambda i: (0, 0)),
            pl.BlockSpec((1, 512), lambda i: (0, 0)),
            pl.BlockSpec((512, F), lambda i: (0, 0)),
            pl.BlockSpec((1, F), lambda i: (0, 0)),
            pl.BlockSpec((F, 2), lambda i: (0, 0)),
            pl.BlockSpec((1, 2), lambda i: (0, 0)),
        ],
        out_specs=pl.BlockSpec((1, 2), lambda i: (0, 0)),
        out_shape=jax.ShapeDtypeStruct((1, 2), jnp.float32),
        scratch_shapes=[pltpu.VMEM((8, F), jnp.float32)],
    )(agg, degi, b2, lw1, lb1, lw2, lb2, lw3, lb3)


def _padv(a, v):
    return jnp.concatenate([a, jnp.full((PE - E,), v, jnp.int32)])


def kernel(fea1, fea2, edge_index1, edge_index2, W1, b1, W2, b2,
           lw1, lb1, lw2, lb2, lw3, lb3):
    s1, d1 = edge_index1[0], edge_index1[1]
    s2, d2 = edge_index2[0], edge_index2[1]

    srcd = jnp.stack([_padv(s1, DUMP), _padv(s2, DUMP)]).reshape(2, NS, CHUNKS, C)
    srcx = jnp.stack([_padv(s1, 0), _padv(s2, 0) + N]).reshape(2, NS, CHUNKS, C)
    dstl = jnp.stack([_padv(d1, DUMP), _padv(d2, DUMP)]).reshape(2, NS, CHUNKS, C)

    zdeg = jnp.zeros((DEGT_T, 16), jnp.float32)
    ones8 = jnp.ones((C, 8), jnp.float32)
    zero8 = jnp.zeros((C, 8), jnp.float32)
    ones_lo = jnp.concatenate([ones8, zero8], axis=1)
    ones_hi = jnp.concatenate([zero8, ones8], axis=1)
    zagg = jnp.zeros((AGGR_T, F), jnp.bfloat16)

    degs = _deg_kernel(srcd, dstl, zdeg, ones_lo, ones_hi)
    dego = jnp.concatenate([degs[0, :N, 0], degs[1, :N, 0]]).reshape(2 * N, 1)
    degi = jnp.concatenate([degs[0, :N, 8], degs[1, :N, 8]]).reshape(2 * N, 1)

    x = jnp.concatenate([fea1, fea2], axis=0)
    u1 = _mm1(x, W1)
    hs1 = _scale1(u1, dego)
    a1 = _agg_kernel(hs1, srcx, dstl, zagg)
    hs2 = _prep2(a1, degi, dego, b1.reshape(1, F), W2)
    a2 = _agg_kernel(hs2, srcx, dstl, zagg)
    return _final(a2, degi, b2.reshape(1, F), lw1,
                  lb1.reshape(1, 512), lw2, lb2.reshape(1, F),
                  lw3, lb3.reshape(1, 2))
